# Initial kernel scaffold; baseline (speedup 1.0000x reference)
import jax
import jax.numpy as jnp
from jax import lax
from jax.experimental import pallas as pl
from jax.experimental.pallas import tpu as pltpu
from jax.experimental.pallas import tpu_sc as plsc

NC = 2
NS = 16
NW = NC * NS

MAXLEN = 200
D = 32

BATCH = 4096
B_TOTAL = BATCH * MAXLEN
PER_W = B_TOTAL // NW
SEQ_PER_CHUNK = 8
CH = SEQ_PER_CHUNK * MAXLEN
N_CHUNKS = PER_W // CH


def _body(x_hbm, tok_hbm, pos_hbm, out_hbm, idx_v, rows_v, pos_v, sem):
    wid = lax.axis_index("s") * NC + lax.axis_index("c")
    base = wid * PER_W

    pltpu.sync_copy(pos_hbm, pos_v)

    def chunk_body(c, _):
        rbase = base + c * CH
        pltpu.sync_copy(x_hbm.at[pl.ds(rbase, CH)], idx_v)
        pltpu.async_copy(tok_hbm.at[idx_v], rows_v, sem).wait()

        def t_loop(t, _):
            p0 = pos_v[t, pl.ds(0, 16)]
            p1 = pos_v[t, pl.ds(16, 16)]

            def s_loop(s, _):
                r = s * MAXLEN + t
                rows_v[r, pl.ds(0, 16)] = rows_v[r, pl.ds(0, 16)] + p0
                rows_v[r, pl.ds(16, 16)] = rows_v[r, pl.ds(16, 16)] + p1
                return 0

            return lax.fori_loop(0, SEQ_PER_CHUNK, s_loop, 0)

        lax.fori_loop(0, MAXLEN, t_loop, 0)

        pltpu.sync_copy(rows_v, out_hbm.at[pl.ds(rbase, CH)])
        return 0

    lax.fori_loop(0, N_CHUNKS, chunk_body, 0)


@jax.jit
def _embed(x_flat, token_table, pos_table):
    mesh = plsc.VectorSubcoreMesh(core_axis_name="c", subcore_axis_name="s")
    return pl.kernel(
        _body,
        out_type=jax.ShapeDtypeStruct((B_TOTAL, D), jnp.float32),
        mesh=mesh,
        scratch_types=[
            pltpu.VMEM((CH,), jnp.int32),
            pltpu.VMEM((CH, D), jnp.float32),
            pltpu.VMEM((MAXLEN, D), jnp.float32),
            pltpu.SemaphoreType.DMA,
        ],
    )(x_flat, token_table, pos_table)


def kernel(x, token_table, pos_table):
    out = _embed(x.reshape(-1).astype(jnp.int32), token_table, pos_table)
    return out.reshape(x.shape[0], x.shape[1], D)


# speedup vs baseline: 4.8652x; 4.8652x over previous
import jax
import jax.numpy as jnp
from jax import lax
from jax.experimental import pallas as pl
from jax.experimental.pallas import tpu as pltpu
from jax.experimental.pallas import tpu_sc as plsc

NC = 2
NS = 16
NW = NC * NS

MAXLEN = 200
D = 32

BATCH = 4096
B_TOTAL = BATCH * MAXLEN
PER_W = B_TOTAL // NW
SEQ_PER_CHUNK = 8
CH = SEQ_PER_CHUNK * MAXLEN
N_CHUNKS = PER_W // CH


def _body(x_hbm, tok_hbm, pos_hbm, out_hbm, idx_v, rows_v, pos_v, sem):
    wid = lax.axis_index("s") * NC + lax.axis_index("c")
    base = wid * PER_W

    pltpu.sync_copy(pos_hbm, pos_v)

    def chunk_body(c, _):
        rbase = base + c * CH
        pltpu.sync_copy(x_hbm.at[pl.ds(rbase, CH)], idx_v)
        pltpu.async_copy(tok_hbm.at[idx_v], rows_v, sem).wait()

        def t_loop(t, _):
            p0 = pos_v[t, pl.ds(0, 16)]
            p1 = pos_v[t, pl.ds(16, 16)]

            def s_loop(s, _):
                r = s * MAXLEN + t
                rows_v[r, pl.ds(0, 16)] = rows_v[r, pl.ds(0, 16)] + p0
                rows_v[r, pl.ds(16, 16)] = rows_v[r, pl.ds(16, 16)] + p1
                return 0

            return lax.fori_loop(0, SEQ_PER_CHUNK, s_loop, 0)

        lax.fori_loop(0, MAXLEN, t_loop, 0)

        pltpu.sync_copy(rows_v, out_hbm.at[pl.ds(rbase, CH)])
        return 0

    lax.fori_loop(0, N_CHUNKS, chunk_body, 0)


@jax.jit
def _embed(x_flat, token_table, pos_table):
    mesh = plsc.VectorSubcoreMesh(core_axis_name="c", subcore_axis_name="s")
    return pl.kernel(
        _body,
        out_type=jax.ShapeDtypeStruct((B_TOTAL, D), jnp.float32),
        mesh=mesh,
        scratch_types=[
            pltpu.VMEM((CH,), jnp.int32),
            pltpu.VMEM((CH, D), jnp.float32),
            pltpu.VMEM((MAXLEN, D), jnp.float32),
            pltpu.SemaphoreType.DMA,
        ],
        compiler_params=pltpu.CompilerParams(use_tc_tiling_on_sc=False),
    )(x_flat, token_table, pos_table)


def kernel(x, token_table, pos_table):
    out = _embed(x.reshape(-1).astype(jnp.int32), token_table, pos_table)
    return out.reshape(x.shape[0], x.shape[1], D)


# speedup vs baseline: 5.2563x; 1.0804x over previous
import jax
import jax.numpy as jnp
from jax import lax
from jax.experimental import pallas as pl
from jax.experimental.pallas import tpu as pltpu
from jax.experimental.pallas import tpu_sc as plsc

NC = 2
NS = 16
NW = NC * NS

MAXLEN = 200
D = 32

BATCH = 4096
B_TOTAL = BATCH * MAXLEN
PER_W = B_TOTAL // NW
SEQ_PER_CHUNK = 4
CH = SEQ_PER_CHUNK * MAXLEN
N_CHUNKS = PER_W // CH


def _body(x_hbm, tok_hbm, pos_hbm, out_hbm, idx_v, rows0, rows1, pos_v,
          sem_g0, sem_g1, sem_s0, sem_s1):
    wid = lax.axis_index("s") * NC + lax.axis_index("c")
    base = wid * PER_W

    rows = (rows0, rows1)
    sem_g = (sem_g0, sem_g1)
    sem_s = (sem_s0, sem_s1)

    pltpu.sync_copy(x_hbm.at[wid], idx_v)
    pltpu.sync_copy(pos_hbm, pos_v)

    def add_pos(buf):
        def t_loop(t, _):
            p0 = pos_v[t, pl.ds(0, 16)]
            p1 = pos_v[t, pl.ds(16, 16)]
            for s in range(SEQ_PER_CHUNK):
                r = s * MAXLEN + t
                buf[r, pl.ds(0, 16)] = buf[r, pl.ds(0, 16)] + p0
                buf[r, pl.ds(16, 16)] = buf[r, pl.ds(16, 16)] + p1
            return 0

        lax.fori_loop(0, MAXLEN, t_loop, 0)

    pltpu.async_copy(tok_hbm.at[idx_v.at[0]], rows0, sem_g0)

    def pair_body(j, _):
        for b in range(2):
            c = 2 * j + b
            nb = 1 - b
            pltpu.make_async_copy(tok_hbm.at[idx_v.at[c]], rows[b],
                                  sem_g[b]).wait()
            @pl.when(c >= 1)
            def _():
                pltpu.make_async_copy(
                    rows[nb], out_hbm.at[pl.ds(base + (c - 1) * CH, CH)],
                    sem_s[nb]).wait()

            @pl.when(c + 1 < N_CHUNKS)
            def _():
                pltpu.async_copy(tok_hbm.at[idx_v.at[c + 1]], rows[nb],
                                 sem_g[nb])

            add_pos(rows[b])
            pltpu.async_copy(rows[b],
                             out_hbm.at[pl.ds(base + c * CH, CH)], sem_s[b])
        return 0

    lax.fori_loop(0, N_CHUNKS // 2, pair_body, 0)

    last = N_CHUNKS - 1
    pltpu.make_async_copy(rows[1], out_hbm.at[pl.ds(base + last * CH, CH)],
                          sem_s[1]).wait()


@jax.jit
def _embed(x_split, token_table, pos_table):
    mesh = plsc.VectorSubcoreMesh(core_axis_name="c", subcore_axis_name="s")
    return pl.kernel(
        _body,
        out_type=jax.ShapeDtypeStruct((B_TOTAL, D), jnp.float32),
        mesh=mesh,
        scratch_types=[
            pltpu.VMEM((N_CHUNKS, CH), jnp.int32),
            pltpu.VMEM((CH, D), jnp.float32),
            pltpu.VMEM((CH, D), jnp.float32),
            pltpu.VMEM((MAXLEN, D), jnp.float32),
            pltpu.SemaphoreType.DMA,
            pltpu.SemaphoreType.DMA,
            pltpu.SemaphoreType.DMA,
            pltpu.SemaphoreType.DMA,
        ],
        compiler_params=pltpu.CompilerParams(use_tc_tiling_on_sc=False),
    )(x_split, token_table, pos_table)


def kernel(x, token_table, pos_table):
    x_split = x.reshape(NW, N_CHUNKS, CH).astype(jnp.int32)
    out = _embed(x_split, token_table, pos_table)
    return out.reshape(x.shape[0], x.shape[1], D)
